# baseline (speedup 1.0000x reference)
import jax
import jax.numpy as jnp
from jax.experimental import pallas as pl
from jax.experimental.pallas import tpu as pltpu


def _pos_kernel(row_ref, col_ref, out_ref, big, sems):
    h, d = row_ref.shape
    w = col_ref.shape[0]
    hw = h * w
    b = out_ref.shape[0]
    lane = jax.lax.broadcasted_iota(jnp.int32, (w, hw), 1)
    sub_w = jax.lax.broadcasted_iota(jnp.int32, (w, hw), 0)
    sx = jnp.where(lane % w == sub_w, 1.0, 0.0).astype(jnp.float32)
    lane_h = jax.lax.broadcasted_iota(jnp.int32, (h, hw), 1)
    sub_h = jax.lax.broadcasted_iota(jnp.int32, (h, hw), 0)
    sy = jnp.where(lane_h // w == sub_h, 1.0, 0.0).astype(jnp.float32)
    dn = (((0,), (0,)), ((), ()))
    xm = jax.lax.dot_general(
        col_ref[...], sx, dn, preferred_element_type=jnp.float32)
    ym = jax.lax.dot_general(
        row_ref[...], sy, dn, preferred_element_type=jnp.float32)
    big[0, :d, :] = xm
    big[0, d:, :] = ym
    pat = big[0]
    for i in range(1, b):
        big[i] = pat
    half = b // 2
    c0 = pltpu.make_async_copy(
        big.at[pl.ds(0, half)], out_ref.at[pl.ds(0, half)], sems.at[0])
    c1 = pltpu.make_async_copy(
        big.at[pl.ds(half, half)], out_ref.at[pl.ds(half, half)], sems.at[1])
    c0.start(priority=0)
    c1.start(priority=1)
    c0.wait()
    c1.wait()


def kernel(pixel_values, row_embedding, column_embedding):
    b = pixel_values.shape[0]
    h, w = pixel_values.shape[-2], pixel_values.shape[-1]
    d = row_embedding.shape[-1]
    row = row_embedding[:h]
    col = column_embedding[:w]
    out = pl.pallas_call(
        _pos_kernel,
        in_specs=[
            pl.BlockSpec((h, d), lambda: (0, 0)),
            pl.BlockSpec((w, d), lambda: (0, 0)),
        ],
        out_specs=pl.BlockSpec(memory_space=pl.ANY),
        out_shape=jax.ShapeDtypeStruct((b, 2 * d, h * w), jnp.float32),
        scratch_shapes=[
            pltpu.VMEM((b, 2 * d, h * w), jnp.float32),
            pltpu.SemaphoreType.DMA((2,)),
        ],
    )(row, col)
    return out.reshape(b, 2 * d, h, w)
